# Initial kernel scaffold; baseline (speedup 1.0000x reference)
import jax
import jax.numpy as jnp
from jax.experimental import pallas as pl


def kernel(x, pos_feat, topo_prompt, edge_index, edge_weight, params):
    raise NotImplementedError("write your pallas kernel here")


# speedup vs baseline: 8.2438x; 8.2438x over previous
import functools

import jax
import jax.numpy as jnp
from jax import lax
from jax.experimental import pallas as pl
from jax.experimental.pallas import tpu as pltpu
from jax.experimental.pallas import tpu_sc as plsc

N = 10000
NPAD = 10240
E = 320000
NW = 32
EPT = E // NW
CHUNK = 125
NCHUNK = EPT // CHUNK
RPT = NPAD // 16

@functools.lru_cache(maxsize=None)
def _mesh():
  return plsc.VectorSubcoreMesh(
      core_axis_name="c", subcore_axis_name="s", num_cores=2, num_subcores=16)


def _sc_degree(row3, ew3, zrow):
  def body(row_hbm, ew_hbm, z_hbm, out_hbm, row_v, ew_v, deg_sh):
    c = lax.axis_index("c")
    s = lax.axis_index("s")
    wid = c * 16 + s
    pltpu.sync_copy(row_hbm.at[wid], row_v)
    pltpu.sync_copy(ew_hbm.at[wid], ew_v)
    pltpu.sync_copy(z_hbm, deg_sh.at[pl.ds(s * RPT, RPT)])
    plsc.subcore_barrier()

    def chunk(j, carry):
      pltpu.sync_copy(ew_v.at[j], deg_sh.at[row_v.at[j]], add=True)
      return carry

    lax.fori_loop(0, NCHUNK, chunk, 0)
    plsc.subcore_barrier()
    pltpu.sync_copy(deg_sh.at[pl.ds(s * RPT, RPT)],
                    out_hbm.at[c, pl.ds(s * RPT, RPT)])

  return pl.kernel(
      body,
      out_type=jax.ShapeDtypeStruct((2, NPAD), jnp.float32),
      mesh=_mesh(),
      compiler_params=pltpu.CompilerParams(needs_layout_passes=False),
      scratch_types=[
          pltpu.VMEM((NCHUNK, CHUNK), jnp.int32),
          pltpu.VMEM((NCHUNK, CHUNK), jnp.float32),
          pltpu.VMEM_SHARED((NPAD,), jnp.float32),
      ],
  )(row3, ew3, zrow)


SEC = 16


def _sc_aggregate(row3, col3, ew3, g, z2):
  def body(row_hbm, col_hbm, ew_hbm, g_hbm, z_hbm, out_hbm,
           row_v, col_v, ew_v, gbuf, sem, acc_sh):
    c = lax.axis_index("c")
    s = lax.axis_index("s")
    wid = c * 16 + s
    pltpu.sync_copy(z_hbm, acc_sh.at[pl.ds(s * RPT, RPT)])
    plsc.subcore_barrier()

    def chunk(j, carry):
      sec = j // SEC
      jl = j % SEC

      @pl.when(jl == 0)
      def _():
        pltpu.sync_copy(row_hbm.at[wid, pl.ds(sec * SEC, SEC)], row_v)
        pltpu.sync_copy(col_hbm.at[wid, pl.ds(sec * SEC, SEC)], col_v)
        pltpu.sync_copy(ew_hbm.at[wid, pl.ds(sec * SEC, SEC)], ew_v)

      pltpu.async_copy(g_hbm.at[col_v.at[jl]], gbuf, sem).wait()
      jv = jnp.zeros((16,), jnp.int32) + jl
      ivec = jnp.zeros((16,), jnp.int32)
      one = jnp.ones((16,), jnp.int32)
      for e in range(CHUNK):
        sp = plsc.load_gather(ew_v, [jv, ivec])
        ivec = ivec + one
        for d in range(8):
          ds_ = pl.ds(d * 16, 16)
          gbuf[e, ds_] = gbuf[e, ds_] * sp
      pltpu.sync_copy(gbuf, acc_sh.at[row_v.at[jl]], add=True)
      return carry

    lax.fori_loop(0, NCHUNK, chunk, 0)
    plsc.subcore_barrier()
    pltpu.sync_copy(acc_sh.at[pl.ds(s * RPT, RPT)],
                    out_hbm.at[c, pl.ds(s * RPT, RPT)])

  return pl.kernel(
      body,
      out_type=jax.ShapeDtypeStruct((2, NPAD, 128), jnp.float32),
      mesh=_mesh(),
      compiler_params=pltpu.CompilerParams(needs_layout_passes=False),
      scratch_types=[
          pltpu.VMEM((SEC, CHUNK), jnp.int32),
          pltpu.VMEM((SEC, CHUNK), jnp.int32),
          pltpu.VMEM((SEC, CHUNK), jnp.float32),
          pltpu.VMEM((CHUNK, 128), jnp.float32),
          pltpu.SemaphoreType.DMA,
          pltpu.VMEM_SHARED((NPAD, 128), jnp.float32),
      ],
  )(row3, col3, ew3, g, z2)


def _tc_scale(hl0a, hl0b, dis_col):
  def body(a_ref, b_ref, d_ref, oa_ref, ob_ref):
    oa_ref[...] = a_ref[...] * d_ref[...]
    ob_ref[...] = b_ref[...] * d_ref[...]

  return pl.pallas_call(
      body,
      grid=(N // _R,),
      in_specs=[_rows(128), _rows(128), _rows(1)],
      out_specs=[_rows(128), _rows(128)],
      out_shape=[
          jax.ShapeDtypeStruct((N, 128), jnp.float32),
          jax.ShapeDtypeStruct((N, 128), jnp.float32),
      ],
  )(hl0a, hl0b, dis_col)


def _ln(x, g, b, eps=1e-5):
  m = jnp.mean(x, axis=-1, keepdims=True)
  v = jnp.mean((x - m) * (x - m), axis=-1, keepdims=True)
  return (x - m) * lax.rsqrt(v + eps) * g + b


def _dot(a, b):
  return jnp.dot(a, b, preferred_element_type=jnp.float32)


_R = 1000


def _full(shape):
  return pl.BlockSpec(shape, lambda i: tuple(0 for _ in shape))


def _rows(width):
  return pl.BlockSpec((_R, width), lambda i: (i, 0))


def _parts():
  return pl.BlockSpec((2, _R, 128), lambda i: (0, i, 0))


def _tc_stage_a(x, pos_feat, p):
  def body(x_ref, pos_ref, w1_ref, b1_ref, w2_ref, b2_ref, wg_ref, bg_ref,
           wr_ref, hla_ref, hlb_ref, res_ref):
    pe = jnp.maximum(_dot(pos_ref[...], w1_ref[...]) + b1_ref[...], 0.0)
    pe = _dot(pe, w2_ref[...]) + b2_ref[...]
    h = jnp.concatenate([x_ref[...], pe], axis=1)
    hl = _dot(h, wg_ref[...]) + bg_ref[...]
    hla_ref[...] = hl[:, :128]
    hlb_ref[...] = hl[:, 128:]
    res_ref[...] = _dot(h, wr_ref[...])

  return pl.pallas_call(
      body,
      grid=(N // _R,),
      in_specs=[
          _rows(128), _rows(16),
          _full((16, 16)), _full((1, 16)),
          _full((16, 16)), _full((1, 16)),
          _full((144, 256)), _full((1, 256)),
          _full((144, 256)),
      ],
      out_specs=[_rows(128), _rows(128), _rows(256)],
      out_shape=[
          jax.ShapeDtypeStruct((N, 128), jnp.float32),
          jax.ShapeDtypeStruct((N, 128), jnp.float32),
          jax.ShapeDtypeStruct((N, 256), jnp.float32),
      ],
  )(x, pos_feat,
    p["pos1_W"].T, p["pos1_b"][None], p["pos2_W"].T, p["pos2_b"][None],
    p["gcn0_W"].T, p["gcn0_b"][None], p["res0_W"].T)


def _tc_stage_b(agg0a, agg0b, hl0a, hl0b, res0, dis_col, p):
  def body(pa_ref, pb_ref, hla_ref, hlb_ref, res_ref, dis_ref, g0_ref, b0_ref,
           wg1_ref, bg1_ref, wr1_ref, h1_ref, hlin1_ref, res1_ref):
    a = jnp.concatenate([pa_ref[0] + pa_ref[1], pb_ref[0] + pb_ref[1]], axis=1)
    hls = jnp.concatenate([hla_ref[...], hlb_ref[...]], axis=1)
    agg = (a + hls) * dis_ref[...] + 0.2 * res_ref[...]
    h1 = jnp.maximum(_ln(agg, g0_ref[...], b0_ref[...]), 0.0)
    h1_ref[...] = h1
    hlin1_ref[...] = (_dot(h1, wg1_ref[...]) + bg1_ref[...]) * dis_ref[...]
    res1_ref[...] = _dot(h1, wr1_ref[...])

  return pl.pallas_call(
      body,
      grid=(N // _R,),
      in_specs=[
          _parts(), _parts(), _rows(128), _rows(128), _rows(256), _rows(1),
          _full((1, 256)), _full((1, 256)),
          _full((256, 128)), _full((1, 128)), _full((256, 128)),
      ],
      out_specs=[_rows(256), _rows(128), _rows(128)],
      out_shape=[
          jax.ShapeDtypeStruct((N, 256), jnp.float32),
          jax.ShapeDtypeStruct((N, 128), jnp.float32),
          jax.ShapeDtypeStruct((N, 128), jnp.float32),
      ],
  )(agg0a, agg0b, hl0a, hl0b, res0, dis_col,
    p["ln0_g"][None], p["ln0_b"][None],
    p["gcn1_W"].T, p["gcn1_b"][None], p["res1_W"].T)


def _tc_stage_c(agg1, hlin1, res1, dis_col, h1, topo, p):
  def body(q_ref, hlin1_ref, res1_ref, dis_ref, h1_ref, topo_ref,
           g1_ref, b1_ref, wjk_ref, bjk_ref, gjk_ref, bjkl_ref,
           wa1_ref, ba1_ref, wa2_ref, ba2_ref, gad_ref, bad_ref, sc_ref,
           wop_ref, bop_ref, gop_ref, bopl_ref,
           zv_ref, zs_ref, rs_ref):
    agg = (q_ref[0] + q_ref[1] + hlin1_ref[...]) * dis_ref[...] \
        + 0.2 * res1_ref[...]
    h2 = jnp.maximum(_ln(agg, g1_ref[...], b1_ref[...]), 0.0)
    jk = jnp.concatenate([h1_ref[...], h2], axis=1)
    zs = _ln(jnp.maximum(_dot(jk, wjk_ref[...]) + bjk_ref[...], 0.0),
             gjk_ref[...], bjkl_ref[...])
    t = jnp.concatenate([zs, topo_ref[...]], axis=1)
    t = jnp.maximum(_dot(t, wa1_ref[...]) + ba1_ref[...], 0.0)
    t = _ln(_dot(t, wa2_ref[...]) + ba2_ref[...], gad_ref[...], bad_ref[...])
    rs = sc_ref[0, 0] * t
    z = jnp.maximum(_dot(zs + rs, wop_ref[...]) + bop_ref[...], 0.0)
    zv_ref[...] = _ln(z, gop_ref[...], bopl_ref[...])
    zs_ref[...] = zs
    rs_ref[...] = rs

  return pl.pallas_call(
      body,
      grid=(N // _R,),
      in_specs=[
          _parts(), _rows(128), _rows(128), _rows(1), _rows(256), _rows(32),
          _full((1, 128)), _full((1, 128)),
          _full((384, 128)), _full((1, 128)), _full((1, 128)), _full((1, 128)),
          _full((160, 128)), _full((1, 128)),
          _full((128, 128)), _full((1, 128)), _full((1, 128)), _full((1, 128)),
          _full((1, 1)),
          _full((128, 128)), _full((1, 128)), _full((1, 128)), _full((1, 128)),
      ],
      out_specs=[_rows(128), _rows(128), _rows(128)],
      out_shape=[
          jax.ShapeDtypeStruct((N, 128), jnp.float32),
          jax.ShapeDtypeStruct((N, 128), jnp.float32),
          jax.ShapeDtypeStruct((N, 128), jnp.float32),
      ],
  )(agg1, hlin1, res1, dis_col, h1, topo,
    p["ln1_g"][None], p["ln1_b"][None],
    p["jk_W"].T, p["jk_b"][None], p["jk_ln_g"][None], p["jk_ln_b"][None],
    p["ad1_W"].T, p["ad1_b"][None],
    p["ad2_W"].T, p["ad2_b"][None], p["ad_ln_g"][None], p["ad_ln_b"][None],
    p["ad_scale"].reshape(1, 1),
    p["op_W"].T, p["op_b"][None], p["op_ln_g"][None], p["op_ln_b"][None])


def kernel(x, pos_feat, topo_prompt, edge_index, edge_weight, params):
  p = params
  row3 = edge_index[0].reshape(NW, NCHUNK, CHUNK)
  col3 = edge_index[1].reshape(NW, NCHUNK, CHUNK)
  ew3 = edge_weight.reshape(NW, NCHUNK, CHUNK)
  zrow = jnp.zeros((RPT,), jnp.float32)
  z2 = jnp.zeros((RPT, 128), jnp.float32)

  degp = _sc_degree(row3, ew3, zrow)
  deg = degp[0] + degp[1] + 1.0
  dis_col = deg[:N, None] ** -0.5

  hl0a, hl0b, res0 = _tc_stage_a(x, pos_feat, p)
  hl0as, hl0bs = _tc_scale(hl0a, hl0b, dis_col)
  agg0a = _sc_aggregate(row3, col3, ew3, hl0as, z2)
  agg0b = _sc_aggregate(row3, col3, ew3, hl0bs, z2)
  h1, hlin1s, res1 = _tc_stage_b(agg0a, agg0b, hl0as, hl0bs, res0, dis_col, p)
  agg1 = _sc_aggregate(row3, col3, ew3, hlin1s, z2)
  return _tc_stage_c(agg1, hlin1s, res1, dis_col, h1, topo_prompt, p)


# speedup vs baseline: 9.0660x; 1.0997x over previous
import functools

import jax
import jax.numpy as jnp
from jax import lax
from jax.experimental import pallas as pl
from jax.experimental.pallas import tpu as pltpu
from jax.experimental.pallas import tpu_sc as plsc

N = 10000
NPAD = 10240
E = 320000
NW = 32
EPT = E // NW
CHUNK = 50
NCHUNK = EPT // CHUNK
RPT = NPAD // 16

@functools.lru_cache(maxsize=None)
def _mesh():
  return plsc.VectorSubcoreMesh(
      core_axis_name="c", subcore_axis_name="s", num_cores=2, num_subcores=16)


def _sc_degree(row3, ew3, zrow):
  def body(row_hbm, ew_hbm, z_hbm, out_hbm, row_v, ew_v, deg_sh):
    c = lax.axis_index("c")
    s = lax.axis_index("s")
    wid = c * 16 + s
    pltpu.sync_copy(row_hbm.at[wid], row_v)
    pltpu.sync_copy(ew_hbm.at[wid], ew_v)
    pltpu.sync_copy(z_hbm, deg_sh.at[pl.ds(s * RPT, RPT)])
    plsc.subcore_barrier()

    def chunk(j, carry):
      pltpu.sync_copy(ew_v.at[j], deg_sh.at[row_v.at[j]], add=True)
      return carry

    lax.fori_loop(0, NCHUNK, chunk, 0)
    plsc.subcore_barrier()
    pltpu.sync_copy(deg_sh.at[pl.ds(s * RPT, RPT)],
                    out_hbm.at[c, pl.ds(s * RPT, RPT)])

  return pl.kernel(
      body,
      out_type=jax.ShapeDtypeStruct((2, NPAD), jnp.float32),
      mesh=_mesh(),
      compiler_params=pltpu.CompilerParams(needs_layout_passes=False),
      scratch_types=[
          pltpu.VMEM((NCHUNK, CHUNK), jnp.int32),
          pltpu.VMEM((NCHUNK, CHUNK), jnp.float32),
          pltpu.VMEM_SHARED((NPAD,), jnp.float32),
      ],
  )(row3, ew3, zrow)


SEC = 8
NSECT = NCHUNK // SEC
NP = NCHUNK // 2


def _sc_aggregate(row3, col3, ew3, g, z2):
  def body(row_hbm, col_hbm, ew_hbm, g_hbm, z_hbm, out_hbm,
           row_v, col_v, ew_v, gbuf0, gbuf1, gsem0, gsem1, ssem0, ssem1,
           acc_sh):
    c = lax.axis_index("c")
    s = lax.axis_index("s")
    wid = c * 16 + s
    pltpu.sync_copy(z_hbm, acc_sh.at[pl.ds(s * RPT, RPT)])
    pltpu.sync_copy(row_hbm.at[wid, pl.ds(0, SEC)], row_v.at[0])
    pltpu.sync_copy(col_hbm.at[wid, pl.ds(0, SEC)], col_v.at[0])
    pltpu.sync_copy(ew_hbm.at[wid, pl.ds(0, SEC)], ew_v.at[0])
    plsc.subcore_barrier()
    pltpu.async_copy(g_hbm.at[col_v.at[0, 0]], gbuf0, gsem0)

    def scale(gbuf, sb, jl):
      sbv = jnp.zeros((16,), jnp.int32) + sb
      jv = jnp.zeros((16,), jnp.int32) + jl
      ivec = jnp.zeros((16,), jnp.int32)
      one = jnp.ones((16,), jnp.int32)
      for e in range(CHUNK):
        sp = plsc.load_gather(ew_v, [sbv, jv, ivec])
        ivec = ivec + one
        for d in range(8):
          ds_ = pl.ds(d * 16, 16)
          gbuf[e, ds_] = gbuf[e, ds_] * sp

    def swait(gbuf, sem):
      pltpu.make_async_copy(gbuf, acc_sh.at[row_v.at[0, 0]], sem).wait()

    def pair(p, carry):
      j0 = 2 * p
      sec = j0 // SEC
      sb = lax.rem(sec, 2)
      jl0 = lax.rem(j0, SEC)
      jl1 = jl0 + 1

      @pl.when(p > 0)
      def _():
        swait(gbuf1, ssem1)

      @pl.when(jl0 == 0)
      def _():
        @pl.when(sec + 1 < NSECT)
        def _():
          off = (sec + 1) * SEC
          nsb = 1 - sb
          pltpu.sync_copy(row_hbm.at[wid, pl.ds(off, SEC)], row_v.at[nsb])
          pltpu.sync_copy(col_hbm.at[wid, pl.ds(off, SEC)], col_v.at[nsb])
          pltpu.sync_copy(ew_hbm.at[wid, pl.ds(off, SEC)], ew_v.at[nsb])

      pltpu.async_copy(g_hbm.at[col_v.at[sb, jl1]], gbuf1, gsem1)
      pltpu.make_async_copy(g_hbm.at[col_v.at[sb, jl0]], gbuf0, gsem0).wait()
      scale(gbuf0, sb, jl0)
      pltpu.async_copy(gbuf0, acc_sh.at[row_v.at[sb, jl0]], ssem0, add=True)

      pltpu.make_async_copy(g_hbm.at[col_v.at[sb, jl1]], gbuf1, gsem1).wait()
      scale(gbuf1, sb, jl1)

      @pl.when(p + 1 < NP)
      def _():
        swait(gbuf0, ssem0)
        j2 = j0 + 2
        sb2 = lax.rem(j2 // SEC, 2)
        jl2 = lax.rem(j2, SEC)
        pltpu.async_copy(g_hbm.at[col_v.at[sb2, jl2]], gbuf0, gsem0)

      pltpu.async_copy(gbuf1, acc_sh.at[row_v.at[sb, jl1]], ssem1, add=True)
      return carry

    lax.fori_loop(0, NP, pair, 0)
    swait(gbuf0, ssem0)
    swait(gbuf1, ssem1)
    plsc.subcore_barrier()
    pltpu.sync_copy(acc_sh.at[pl.ds(s * RPT, RPT)],
                    out_hbm.at[c, pl.ds(s * RPT, RPT)])

  return pl.kernel(
      body,
      out_type=jax.ShapeDtypeStruct((2, NPAD, 128), jnp.float32),
      mesh=_mesh(),
      compiler_params=pltpu.CompilerParams(needs_layout_passes=False),
      scratch_types=[
          pltpu.VMEM((2, SEC, CHUNK), jnp.int32),
          pltpu.VMEM((2, SEC, CHUNK), jnp.int32),
          pltpu.VMEM((2, SEC, CHUNK), jnp.float32),
          pltpu.VMEM((CHUNK, 128), jnp.float32),
          pltpu.VMEM((CHUNK, 128), jnp.float32),
          pltpu.SemaphoreType.DMA,
          pltpu.SemaphoreType.DMA,
          pltpu.SemaphoreType.DMA,
          pltpu.SemaphoreType.DMA,
          pltpu.VMEM_SHARED((NPAD, 128), jnp.float32),
      ],
  )(row3, col3, ew3, g, z2)


def _tc_scale(hl0a, hl0b, dis_col):
  def body(a_ref, b_ref, d_ref, oa_ref, ob_ref):
    oa_ref[...] = a_ref[...] * d_ref[...]
    ob_ref[...] = b_ref[...] * d_ref[...]

  return pl.pallas_call(
      body,
      grid=(N // _R,),
      in_specs=[_rows(128), _rows(128), _rows(1)],
      out_specs=[_rows(128), _rows(128)],
      out_shape=[
          jax.ShapeDtypeStruct((N, 128), jnp.float32),
          jax.ShapeDtypeStruct((N, 128), jnp.float32),
      ],
  )(hl0a, hl0b, dis_col)


def _ln(x, g, b, eps=1e-5):
  m = jnp.mean(x, axis=-1, keepdims=True)
  v = jnp.mean((x - m) * (x - m), axis=-1, keepdims=True)
  return (x - m) * lax.rsqrt(v + eps) * g + b


def _dot(a, b):
  return jnp.dot(a, b, preferred_element_type=jnp.float32)


_R = 1000


def _full(shape):
  return pl.BlockSpec(shape, lambda i: tuple(0 for _ in shape))


def _rows(width):
  return pl.BlockSpec((_R, width), lambda i: (i, 0))


def _parts():
  return pl.BlockSpec((2, _R, 128), lambda i: (0, i, 0))


def _tc_stage_a(x, pos_feat, p):
  def body(x_ref, pos_ref, w1_ref, b1_ref, w2_ref, b2_ref, wg_ref, bg_ref,
           wr_ref, hla_ref, hlb_ref, res_ref):
    pe = jnp.maximum(_dot(pos_ref[...], w1_ref[...]) + b1_ref[...], 0.0)
    pe = _dot(pe, w2_ref[...]) + b2_ref[...]
    h = jnp.concatenate([x_ref[...], pe], axis=1)
    hl = _dot(h, wg_ref[...]) + bg_ref[...]
    hla_ref[...] = hl[:, :128]
    hlb_ref[...] = hl[:, 128:]
    res_ref[...] = _dot(h, wr_ref[...])

  return pl.pallas_call(
      body,
      grid=(N // _R,),
      in_specs=[
          _rows(128), _rows(16),
          _full((16, 16)), _full((1, 16)),
          _full((16, 16)), _full((1, 16)),
          _full((144, 256)), _full((1, 256)),
          _full((144, 256)),
      ],
      out_specs=[_rows(128), _rows(128), _rows(256)],
      out_shape=[
          jax.ShapeDtypeStruct((N, 128), jnp.float32),
          jax.ShapeDtypeStruct((N, 128), jnp.float32),
          jax.ShapeDtypeStruct((N, 256), jnp.float32),
      ],
  )(x, pos_feat,
    p["pos1_W"].T, p["pos1_b"][None], p["pos2_W"].T, p["pos2_b"][None],
    p["gcn0_W"].T, p["gcn0_b"][None], p["res0_W"].T)


def _tc_stage_b(agg0a, agg0b, hl0a, hl0b, res0, dis_col, p):
  def body(pa_ref, pb_ref, hla_ref, hlb_ref, res_ref, dis_ref, g0_ref, b0_ref,
           wg1_ref, bg1_ref, wr1_ref, h1_ref, hlin1_ref, res1_ref):
    a = jnp.concatenate([pa_ref[0] + pa_ref[1], pb_ref[0] + pb_ref[1]], axis=1)
    hls = jnp.concatenate([hla_ref[...], hlb_ref[...]], axis=1)
    agg = (a + hls) * dis_ref[...] + 0.2 * res_ref[...]
    h1 = jnp.maximum(_ln(agg, g0_ref[...], b0_ref[...]), 0.0)
    h1_ref[...] = h1
    hlin1_ref[...] = (_dot(h1, wg1_ref[...]) + bg1_ref[...]) * dis_ref[...]
    res1_ref[...] = _dot(h1, wr1_ref[...])

  return pl.pallas_call(
      body,
      grid=(N // _R,),
      in_specs=[
          _parts(), _parts(), _rows(128), _rows(128), _rows(256), _rows(1),
          _full((1, 256)), _full((1, 256)),
          _full((256, 128)), _full((1, 128)), _full((256, 128)),
      ],
      out_specs=[_rows(256), _rows(128), _rows(128)],
      out_shape=[
          jax.ShapeDtypeStruct((N, 256), jnp.float32),
          jax.ShapeDtypeStruct((N, 128), jnp.float32),
          jax.ShapeDtypeStruct((N, 128), jnp.float32),
      ],
  )(agg0a, agg0b, hl0a, hl0b, res0, dis_col,
    p["ln0_g"][None], p["ln0_b"][None],
    p["gcn1_W"].T, p["gcn1_b"][None], p["res1_W"].T)


def _tc_stage_c(agg1, hlin1, res1, dis_col, h1, topo, p):
  def body(q_ref, hlin1_ref, res1_ref, dis_ref, h1_ref, topo_ref,
           g1_ref, b1_ref, wjk_ref, bjk_ref, gjk_ref, bjkl_ref,
           wa1_ref, ba1_ref, wa2_ref, ba2_ref, gad_ref, bad_ref, sc_ref,
           wop_ref, bop_ref, gop_ref, bopl_ref,
           zv_ref, zs_ref, rs_ref):
    agg = (q_ref[0] + q_ref[1] + hlin1_ref[...]) * dis_ref[...] \
        + 0.2 * res1_ref[...]
    h2 = jnp.maximum(_ln(agg, g1_ref[...], b1_ref[...]), 0.0)
    jk = jnp.concatenate([h1_ref[...], h2], axis=1)
    zs = _ln(jnp.maximum(_dot(jk, wjk_ref[...]) + bjk_ref[...], 0.0),
             gjk_ref[...], bjkl_ref[...])
    t = jnp.concatenate([zs, topo_ref[...]], axis=1)
    t = jnp.maximum(_dot(t, wa1_ref[...]) + ba1_ref[...], 0.0)
    t = _ln(_dot(t, wa2_ref[...]) + ba2_ref[...], gad_ref[...], bad_ref[...])
    rs = sc_ref[0, 0] * t
    z = jnp.maximum(_dot(zs + rs, wop_ref[...]) + bop_ref[...], 0.0)
    zv_ref[...] = _ln(z, gop_ref[...], bopl_ref[...])
    zs_ref[...] = zs
    rs_ref[...] = rs

  return pl.pallas_call(
      body,
      grid=(N // _R,),
      in_specs=[
          _parts(), _rows(128), _rows(128), _rows(1), _rows(256), _rows(32),
          _full((1, 128)), _full((1, 128)),
          _full((384, 128)), _full((1, 128)), _full((1, 128)), _full((1, 128)),
          _full((160, 128)), _full((1, 128)),
          _full((128, 128)), _full((1, 128)), _full((1, 128)), _full((1, 128)),
          _full((1, 1)),
          _full((128, 128)), _full((1, 128)), _full((1, 128)), _full((1, 128)),
      ],
      out_specs=[_rows(128), _rows(128), _rows(128)],
      out_shape=[
          jax.ShapeDtypeStruct((N, 128), jnp.float32),
          jax.ShapeDtypeStruct((N, 128), jnp.float32),
          jax.ShapeDtypeStruct((N, 128), jnp.float32),
      ],
  )(agg1, hlin1, res1, dis_col, h1, topo,
    p["ln1_g"][None], p["ln1_b"][None],
    p["jk_W"].T, p["jk_b"][None], p["jk_ln_g"][None], p["jk_ln_b"][None],
    p["ad1_W"].T, p["ad1_b"][None],
    p["ad2_W"].T, p["ad2_b"][None], p["ad_ln_g"][None], p["ad_ln_b"][None],
    p["ad_scale"].reshape(1, 1),
    p["op_W"].T, p["op_b"][None], p["op_ln_g"][None], p["op_ln_b"][None])


def kernel(x, pos_feat, topo_prompt, edge_index, edge_weight, params):
  p = params
  row3 = edge_index[0].reshape(NW, NCHUNK, CHUNK)
  col3 = edge_index[1].reshape(NW, NCHUNK, CHUNK)
  ew3 = edge_weight.reshape(NW, NCHUNK, CHUNK)
  zrow = jnp.zeros((RPT,), jnp.float32)
  z2 = jnp.zeros((RPT, 128), jnp.float32)

  degp = _sc_degree(row3, ew3, zrow)
  deg = degp[0] + degp[1] + 1.0
  dis_col = deg[:N, None] ** -0.5

  hl0a, hl0b, res0 = _tc_stage_a(x, pos_feat, p)
  hl0as, hl0bs = _tc_scale(hl0a, hl0b, dis_col)
  agg0a = _sc_aggregate(row3, col3, ew3, hl0as, z2)
  agg0b = _sc_aggregate(row3, col3, ew3, hl0bs, z2)
  h1, hlin1s, res1 = _tc_stage_b(agg0a, agg0b, hl0as, hl0bs, res0, dis_col, p)
  agg1 = _sc_aggregate(row3, col3, ew3, hlin1s, z2)
  return _tc_stage_c(agg1, hlin1s, res1, dis_col, h1, topo_prompt, p)


# speedup vs baseline: 13.9577x; 1.5396x over previous
import functools

import jax
import jax.numpy as jnp
from jax import lax
from jax.experimental import pallas as pl
from jax.experimental.pallas import tpu as pltpu
from jax.experimental.pallas import tpu_sc as plsc

N = 10000
NPAD = 10240
E = 320000
NW = 32
EPT = E // NW
CHUNK = 50
NCHUNK = EPT // CHUNK
RPT = NPAD // 16

@functools.lru_cache(maxsize=None)
def _mesh():
  return plsc.VectorSubcoreMesh(
      core_axis_name="c", subcore_axis_name="s", num_cores=2, num_subcores=16)


def _sc_degree(row3, ew3, zrow):
  def body(row_hbm, ew_hbm, z_hbm, out_hbm, row_v, ew_v, deg_sh):
    c = lax.axis_index("c")
    s = lax.axis_index("s")
    wid = c * 16 + s
    pltpu.sync_copy(row_hbm.at[wid], row_v)
    pltpu.sync_copy(ew_hbm.at[wid], ew_v)
    pltpu.sync_copy(z_hbm, deg_sh.at[pl.ds(s * RPT, RPT)])
    plsc.subcore_barrier()

    def chunk(j, carry):
      pltpu.sync_copy(ew_v.at[j], deg_sh.at[row_v.at[j]], add=True)
      return carry

    lax.fori_loop(0, NCHUNK, chunk, 0)
    plsc.subcore_barrier()
    pltpu.sync_copy(deg_sh.at[pl.ds(s * RPT, RPT)],
                    out_hbm.at[c, pl.ds(s * RPT, RPT)])

  return pl.kernel(
      body,
      out_type=jax.ShapeDtypeStruct((2, NPAD), jnp.float32),
      mesh=_mesh(),
      compiler_params=pltpu.CompilerParams(needs_layout_passes=False),
      scratch_types=[
          pltpu.VMEM((NCHUNK, CHUNK), jnp.int32),
          pltpu.VMEM((NCHUNK, CHUNK), jnp.float32),
          pltpu.VMEM_SHARED((NPAD,), jnp.float32),
      ],
  )(row3, ew3, zrow)


SEC = 8
NSECT = NCHUNK // SEC
NP = NCHUNK // 2


def _sc_aggregate(row3, col3, ew3, g, z2):
  def body(row_hbm, col_hbm, ew_hbm, g_hbm, z_hbm, out_hbm,
           row_v, col_v, ew_v, gbuf0, gbuf1, gsem0, gsem1, ssem0, ssem1,
           acc_sh):
    c = lax.axis_index("c")
    s = lax.axis_index("s")
    wid = c * 16 + s
    pltpu.sync_copy(z_hbm, acc_sh.at[pl.ds(s * RPT, RPT)])
    pltpu.sync_copy(row_hbm.at[wid, pl.ds(0, SEC)], row_v.at[0])
    pltpu.sync_copy(col_hbm.at[wid, pl.ds(0, SEC)], col_v.at[0])
    pltpu.sync_copy(ew_hbm.at[wid, pl.ds(0, SEC)], ew_v.at[0])
    plsc.subcore_barrier()
    pltpu.async_copy(g_hbm.at[col_v.at[0, 0]], gbuf0, gsem0)

    def scale(gbuf, sb, jl):
      sbv = jnp.zeros((16,), jnp.int32) + sb
      jv = jnp.zeros((16,), jnp.int32) + jl
      sps = [plsc.load_gather(ew_v, [sbv, jv, jnp.full((16,), e, jnp.int32)])
             for e in range(CHUNK)]
      for e in range(CHUNK):
        for d in range(8):
          ds_ = pl.ds(d * 16, 16)
          gbuf[e, ds_] = gbuf[e, ds_] * sps[e]

    def swait(gbuf, sem):
      pltpu.make_async_copy(gbuf, acc_sh.at[row_v.at[0, 0]], sem).wait()

    def pair(p, carry):
      j0 = 2 * p
      sec = j0 // SEC
      sb = lax.rem(sec, 2)
      jl0 = lax.rem(j0, SEC)
      jl1 = jl0 + 1

      @pl.when(p > 0)
      def _():
        swait(gbuf1, ssem1)

      @pl.when(jl0 == 0)
      def _():
        @pl.when(sec + 1 < NSECT)
        def _():
          off = (sec + 1) * SEC
          nsb = 1 - sb
          pltpu.sync_copy(row_hbm.at[wid, pl.ds(off, SEC)], row_v.at[nsb])
          pltpu.sync_copy(col_hbm.at[wid, pl.ds(off, SEC)], col_v.at[nsb])
          pltpu.sync_copy(ew_hbm.at[wid, pl.ds(off, SEC)], ew_v.at[nsb])

      pltpu.async_copy(g_hbm.at[col_v.at[sb, jl1]], gbuf1, gsem1)
      pltpu.make_async_copy(g_hbm.at[col_v.at[sb, jl0]], gbuf0, gsem0).wait()
      scale(gbuf0, sb, jl0)
      pltpu.async_copy(gbuf0, acc_sh.at[row_v.at[sb, jl0]], ssem0, add=True)

      pltpu.make_async_copy(g_hbm.at[col_v.at[sb, jl1]], gbuf1, gsem1).wait()
      scale(gbuf1, sb, jl1)

      @pl.when(p + 1 < NP)
      def _():
        swait(gbuf0, ssem0)
        j2 = j0 + 2
        sb2 = lax.rem(j2 // SEC, 2)
        jl2 = lax.rem(j2, SEC)
        pltpu.async_copy(g_hbm.at[col_v.at[sb2, jl2]], gbuf0, gsem0)

      pltpu.async_copy(gbuf1, acc_sh.at[row_v.at[sb, jl1]], ssem1, add=True)
      return carry

    lax.fori_loop(0, NP, pair, 0)
    swait(gbuf0, ssem0)
    swait(gbuf1, ssem1)
    plsc.subcore_barrier()
    pltpu.sync_copy(acc_sh.at[pl.ds(s * RPT, RPT)],
                    out_hbm.at[c, pl.ds(s * RPT, RPT)])

  return pl.kernel(
      body,
      out_type=jax.ShapeDtypeStruct((2, NPAD, 128), jnp.float32),
      mesh=_mesh(),
      compiler_params=pltpu.CompilerParams(needs_layout_passes=False),
      scratch_types=[
          pltpu.VMEM((2, SEC, CHUNK), jnp.int32),
          pltpu.VMEM((2, SEC, CHUNK), jnp.int32),
          pltpu.VMEM((2, SEC, CHUNK), jnp.float32),
          pltpu.VMEM((CHUNK, 128), jnp.float32),
          pltpu.VMEM((CHUNK, 128), jnp.float32),
          pltpu.SemaphoreType.DMA,
          pltpu.SemaphoreType.DMA,
          pltpu.SemaphoreType.DMA,
          pltpu.SemaphoreType.DMA,
          pltpu.VMEM_SHARED((NPAD, 128), jnp.float32),
      ],
  )(row3, col3, ew3, g, z2)


def _tc_scale(hl0a, hl0b, dis_col):
  def body(a_ref, b_ref, d_ref, oa_ref, ob_ref):
    oa_ref[...] = a_ref[...] * d_ref[...]
    ob_ref[...] = b_ref[...] * d_ref[...]

  return pl.pallas_call(
      body,
      grid=(N // _R,),
      in_specs=[_rows(128), _rows(128), _rows(1)],
      out_specs=[_rows(128), _rows(128)],
      out_shape=[
          jax.ShapeDtypeStruct((N, 128), jnp.float32),
          jax.ShapeDtypeStruct((N, 128), jnp.float32),
      ],
  )(hl0a, hl0b, dis_col)


def _ln(x, g, b, eps=1e-5):
  m = jnp.mean(x, axis=-1, keepdims=True)
  v = jnp.mean((x - m) * (x - m), axis=-1, keepdims=True)
  return (x - m) * lax.rsqrt(v + eps) * g + b


def _dot(a, b):
  return jnp.dot(a, b, preferred_element_type=jnp.float32)


_R = 1000


def _full(shape):
  return pl.BlockSpec(shape, lambda i: tuple(0 for _ in shape))


def _rows(width):
  return pl.BlockSpec((_R, width), lambda i: (i, 0))


def _parts():
  return pl.BlockSpec((2, _R, 128), lambda i: (0, i, 0))


def _tc_stage_a(x, pos_feat, p):
  def body(x_ref, pos_ref, w1_ref, b1_ref, w2_ref, b2_ref, wg_ref, bg_ref,
           wr_ref, hla_ref, hlb_ref, res_ref):
    pe = jnp.maximum(_dot(pos_ref[...], w1_ref[...]) + b1_ref[...], 0.0)
    pe = _dot(pe, w2_ref[...]) + b2_ref[...]
    h = jnp.concatenate([x_ref[...], pe], axis=1)
    hl = _dot(h, wg_ref[...]) + bg_ref[...]
    hla_ref[...] = hl[:, :128]
    hlb_ref[...] = hl[:, 128:]
    res_ref[...] = _dot(h, wr_ref[...])

  return pl.pallas_call(
      body,
      grid=(N // _R,),
      in_specs=[
          _rows(128), _rows(16),
          _full((16, 16)), _full((1, 16)),
          _full((16, 16)), _full((1, 16)),
          _full((144, 256)), _full((1, 256)),
          _full((144, 256)),
      ],
      out_specs=[_rows(128), _rows(128), _rows(256)],
      out_shape=[
          jax.ShapeDtypeStruct((N, 128), jnp.float32),
          jax.ShapeDtypeStruct((N, 128), jnp.float32),
          jax.ShapeDtypeStruct((N, 256), jnp.float32),
      ],
  )(x, pos_feat,
    p["pos1_W"].T, p["pos1_b"][None], p["pos2_W"].T, p["pos2_b"][None],
    p["gcn0_W"].T, p["gcn0_b"][None], p["res0_W"].T)


def _tc_stage_b(agg0a, agg0b, hl0a, hl0b, res0, dis_col, p):
  def body(pa_ref, pb_ref, hla_ref, hlb_ref, res_ref, dis_ref, g0_ref, b0_ref,
           wg1_ref, bg1_ref, wr1_ref, h1_ref, hlin1_ref, res1_ref):
    a = jnp.concatenate([pa_ref[0] + pa_ref[1], pb_ref[0] + pb_ref[1]], axis=1)
    hls = jnp.concatenate([hla_ref[...], hlb_ref[...]], axis=1)
    agg = (a + hls) * dis_ref[...] + 0.2 * res_ref[...]
    h1 = jnp.maximum(_ln(agg, g0_ref[...], b0_ref[...]), 0.0)
    h1_ref[...] = h1
    hlin1_ref[...] = (_dot(h1, wg1_ref[...]) + bg1_ref[...]) * dis_ref[...]
    res1_ref[...] = _dot(h1, wr1_ref[...])

  return pl.pallas_call(
      body,
      grid=(N // _R,),
      in_specs=[
          _parts(), _parts(), _rows(128), _rows(128), _rows(256), _rows(1),
          _full((1, 256)), _full((1, 256)),
          _full((256, 128)), _full((1, 128)), _full((256, 128)),
      ],
      out_specs=[_rows(256), _rows(128), _rows(128)],
      out_shape=[
          jax.ShapeDtypeStruct((N, 256), jnp.float32),
          jax.ShapeDtypeStruct((N, 128), jnp.float32),
          jax.ShapeDtypeStruct((N, 128), jnp.float32),
      ],
  )(agg0a, agg0b, hl0a, hl0b, res0, dis_col,
    p["ln0_g"][None], p["ln0_b"][None],
    p["gcn1_W"].T, p["gcn1_b"][None], p["res1_W"].T)


def _tc_stage_c(agg1, hlin1, res1, dis_col, h1, topo, p):
  def body(q_ref, hlin1_ref, res1_ref, dis_ref, h1_ref, topo_ref,
           g1_ref, b1_ref, wjk_ref, bjk_ref, gjk_ref, bjkl_ref,
           wa1_ref, ba1_ref, wa2_ref, ba2_ref, gad_ref, bad_ref, sc_ref,
           wop_ref, bop_ref, gop_ref, bopl_ref,
           zv_ref, zs_ref, rs_ref):
    agg = (q_ref[0] + q_ref[1] + hlin1_ref[...]) * dis_ref[...] \
        + 0.2 * res1_ref[...]
    h2 = jnp.maximum(_ln(agg, g1_ref[...], b1_ref[...]), 0.0)
    jk = jnp.concatenate([h1_ref[...], h2], axis=1)
    zs = _ln(jnp.maximum(_dot(jk, wjk_ref[...]) + bjk_ref[...], 0.0),
             gjk_ref[...], bjkl_ref[...])
    t = jnp.concatenate([zs, topo_ref[...]], axis=1)
    t = jnp.maximum(_dot(t, wa1_ref[...]) + ba1_ref[...], 0.0)
    t = _ln(_dot(t, wa2_ref[...]) + ba2_ref[...], gad_ref[...], bad_ref[...])
    rs = sc_ref[0, 0] * t
    z = jnp.maximum(_dot(zs + rs, wop_ref[...]) + bop_ref[...], 0.0)
    zv_ref[...] = _ln(z, gop_ref[...], bopl_ref[...])
    zs_ref[...] = zs
    rs_ref[...] = rs

  return pl.pallas_call(
      body,
      grid=(N // _R,),
      in_specs=[
          _parts(), _rows(128), _rows(128), _rows(1), _rows(256), _rows(32),
          _full((1, 128)), _full((1, 128)),
          _full((384, 128)), _full((1, 128)), _full((1, 128)), _full((1, 128)),
          _full((160, 128)), _full((1, 128)),
          _full((128, 128)), _full((1, 128)), _full((1, 128)), _full((1, 128)),
          _full((1, 1)),
          _full((128, 128)), _full((1, 128)), _full((1, 128)), _full((1, 128)),
      ],
      out_specs=[_rows(128), _rows(128), _rows(128)],
      out_shape=[
          jax.ShapeDtypeStruct((N, 128), jnp.float32),
          jax.ShapeDtypeStruct((N, 128), jnp.float32),
          jax.ShapeDtypeStruct((N, 128), jnp.float32),
      ],
  )(agg1, hlin1, res1, dis_col, h1, topo,
    p["ln1_g"][None], p["ln1_b"][None],
    p["jk_W"].T, p["jk_b"][None], p["jk_ln_g"][None], p["jk_ln_b"][None],
    p["ad1_W"].T, p["ad1_b"][None],
    p["ad2_W"].T, p["ad2_b"][None], p["ad_ln_g"][None], p["ad_ln_b"][None],
    p["ad_scale"].reshape(1, 1),
    p["op_W"].T, p["op_b"][None], p["op_ln_g"][None], p["op_ln_b"][None])


def kernel(x, pos_feat, topo_prompt, edge_index, edge_weight, params):
  p = params
  row3 = edge_index[0].reshape(NW, NCHUNK, CHUNK)
  col3 = edge_index[1].reshape(NW, NCHUNK, CHUNK)
  ew3 = edge_weight.reshape(NW, NCHUNK, CHUNK)
  zrow = jnp.zeros((RPT,), jnp.float32)
  z2 = jnp.zeros((RPT, 128), jnp.float32)

  degp = _sc_degree(row3, ew3, zrow)
  deg = degp[0] + degp[1] + 1.0
  dis_col = deg[:N, None] ** -0.5

  hl0a, hl0b, res0 = _tc_stage_a(x, pos_feat, p)
  hl0as, hl0bs = _tc_scale(hl0a, hl0b, dis_col)
  agg0a = _sc_aggregate(row3, col3, ew3, hl0as, z2)
  agg0b = _sc_aggregate(row3, col3, ew3, hl0bs, z2)
  h1, hlin1s, res1 = _tc_stage_b(agg0a, agg0b, hl0as, hl0bs, res0, dis_col, p)
  agg1 = _sc_aggregate(row3, col3, ew3, hlin1s, z2)
  return _tc_stage_c(agg1, hlin1s, res1, dis_col, h1, topo_prompt, p)
